# baseline (speedup 1.0000x reference)
import jax
import jax.numpy as jnp
from jax import lax
from jax.experimental import pallas as pl
from jax.experimental.pallas import tpu as pltpu
from jax.experimental.pallas import tpu_sc as plsc

NUM_ACTIONS = 1000
EMBED = 128
PACKED = EMBED // 2
FEAT = 512
BATCH = 16384

NC, NS = 2, 16
NW = NC * NS
BPW = BATCH // NW
CHUNK = 128
NCHUNK = BPW // CHUNK

BM = 512


def _gather_body(table_hbm, idx_hbm, out_hbm, idx_v, rows_v, sem):
    wid = lax.axis_index("s") * NC + lax.axis_index("c")
    pltpu.sync_copy(idx_hbm.at[wid], idx_v)
    copies = [
        pltpu.async_copy(
            table_hbm.at[idx_v.at[j]],
            rows_v.at[pl.ds(j * CHUNK, CHUNK)],
            sem,
        )
        for j in range(NCHUNK)
    ]
    for c in copies:
        c.wait()
    pltpu.sync_copy(rows_v, out_hbm.at[pl.ds(wid * BPW, BPW)])


_gather = pl.kernel(
    _gather_body,
    out_type=jax.ShapeDtypeStruct((BATCH, PACKED), jnp.int32),
    mesh=plsc.VectorSubcoreMesh(core_axis_name="c", subcore_axis_name="s"),
    scratch_types=[
        pltpu.VMEM((NCHUNK, CHUNK), jnp.int32),
        pltpu.VMEM((BPW, PACKED), jnp.int32),
        pltpu.SemaphoreType.DMA,
    ],
    compiler_params=pltpu.CompilerParams(use_tc_tiling_on_sc=False),
)


def _mlp_body(e_ref, w1_ref, b1_ref, w2_ref, b2_ref, o_ref):
    packed = e_ref[...]
    lo = lax.bitcast_convert_type(packed << 16, jnp.float32)
    hi = lax.bitcast_convert_type(packed & jnp.int32(-65536), jnp.float32)
    e = jnp.concatenate([lo, hi], axis=1).astype(jnp.bfloat16)
    h = jnp.dot(e, w1_ref[...], preferred_element_type=jnp.float32)
    h = h + b1_ref[...]
    h = h * jax.nn.sigmoid(h)
    o = jnp.dot(h.astype(jnp.bfloat16), w2_ref[...],
                preferred_element_type=jnp.float32)
    o_ref[...] = o + b2_ref[...]


def kernel(action_indices, emb_table, W1, b1, W2, b2):
    idx = action_indices.astype(jnp.int32).reshape(NW, NCHUNK, CHUNK)
    table_packed = lax.bitcast_convert_type(
        emb_table.astype(jnp.bfloat16).reshape(NUM_ACTIONS, PACKED, 2),
        jnp.int32,
    )
    W1p = jnp.concatenate([W1[0::2], W1[1::2]], axis=0).astype(jnp.bfloat16)
    embedded = _gather(table_packed, idx)
    out = pl.pallas_call(
        _mlp_body,
        grid=(BATCH // BM,),
        in_specs=[
            pl.BlockSpec((BM, PACKED), lambda i: (i, 0)),
            pl.BlockSpec((EMBED, FEAT), lambda i: (0, 0)),
            pl.BlockSpec((1, FEAT), lambda i: (0, 0)),
            pl.BlockSpec((FEAT, FEAT), lambda i: (0, 0)),
            pl.BlockSpec((1, FEAT), lambda i: (0, 0)),
        ],
        out_specs=pl.BlockSpec((BM, FEAT), lambda i: (i, 0)),
        out_shape=jax.ShapeDtypeStruct((BATCH, FEAT), jnp.float32),
    )(embedded, W1p, b1.reshape(1, FEAT),
      W2.astype(jnp.bfloat16), b2.reshape(1, FEAT))
    return out


# speedup vs baseline: 1.3039x; 1.3039x over previous
import jax
import jax.numpy as jnp
from jax import lax
from jax.experimental import pallas as pl
from jax.experimental.pallas import tpu as pltpu
from jax.experimental.pallas import tpu_sc as plsc

NUM_ACTIONS = 1000
EMBED = 128
FEAT = 512
BATCH = 16384

NC, NS = 2, 16
NW = NC * NS
BPW = BATCH // NW
CHUNK = 64
NCHUNK = BPW // CHUNK


def _mlp_body(t_ref, w1_ref, b1_ref, w2_ref, b2_ref, f_ref):
    e = t_ref[...].astype(jnp.bfloat16)
    h = jnp.dot(e, w1_ref[...], preferred_element_type=jnp.float32)
    h = h + b1_ref[...]
    h = h * jax.nn.sigmoid(h)
    o = jnp.dot(h.astype(jnp.bfloat16), w2_ref[...],
                preferred_element_type=jnp.float32)
    f_ref[...] = o + b2_ref[...]


def _expand_body(feat_hbm, idx_hbm, out_hbm, idx_v, buf0, buf1, sem0, sem1):
    wid = lax.axis_index("s") * NC + lax.axis_index("c")
    base = wid * BPW
    pltpu.sync_copy(idx_hbm.at[wid], idx_v)
    bufs = (buf0, buf1)
    sems = (sem0, sem1)
    copies = [None, None]
    copies[0] = pltpu.async_copy(feat_hbm.at[idx_v.at[0]], buf0, sem0)
    for j in range(NCHUNK):
        if j + 1 < NCHUNK:
            copies[(j + 1) % 2] = pltpu.async_copy(
                feat_hbm.at[idx_v.at[j + 1]], bufs[(j + 1) % 2],
                sems[(j + 1) % 2])
        copies[j % 2].wait()
        pltpu.sync_copy(bufs[j % 2],
                        out_hbm.at[pl.ds(base + j * CHUNK, CHUNK)])


_expand = pl.kernel(
    _expand_body,
    out_type=jax.ShapeDtypeStruct((BATCH, FEAT), jnp.float32),
    mesh=plsc.VectorSubcoreMesh(core_axis_name="c", subcore_axis_name="s"),
    scratch_types=[
        pltpu.VMEM((NCHUNK, CHUNK), jnp.int32),
        pltpu.VMEM((CHUNK, FEAT), jnp.float32),
        pltpu.VMEM((CHUNK, FEAT), jnp.float32),
        pltpu.SemaphoreType.DMA,
        pltpu.SemaphoreType.DMA,
    ],
)


def kernel(action_indices, emb_table, W1, b1, W2, b2):
    idx = action_indices.astype(jnp.int32).reshape(NW, NCHUNK, CHUNK)
    feat = pl.pallas_call(
        _mlp_body,
        in_specs=[
            pl.BlockSpec((NUM_ACTIONS, EMBED), lambda: (0, 0)),
            pl.BlockSpec((EMBED, FEAT), lambda: (0, 0)),
            pl.BlockSpec((1, FEAT), lambda: (0, 0)),
            pl.BlockSpec((FEAT, FEAT), lambda: (0, 0)),
            pl.BlockSpec((1, FEAT), lambda: (0, 0)),
        ],
        out_specs=pl.BlockSpec((NUM_ACTIONS, FEAT), lambda: (0, 0)),
        out_shape=jax.ShapeDtypeStruct((NUM_ACTIONS, FEAT), jnp.float32),
    )(emb_table, W1.astype(jnp.bfloat16), b1.reshape(1, FEAT),
      W2.astype(jnp.bfloat16), b2.reshape(1, FEAT))
    return _expand(feat, idx)


# speedup vs baseline: 1.3326x; 1.0220x over previous
import jax
import jax.numpy as jnp
from jax import lax
from jax.experimental import pallas as pl
from jax.experimental.pallas import tpu as pltpu
from jax.experimental.pallas import tpu_sc as plsc

NUM_ACTIONS = 1000
PAD_ACTIONS = 1024
EMBED = 128
FEAT = 512
BATCH = 16384

MLP_BM = 256

NC, NS = 2, 16
NW = NC * NS
BPW = BATCH // NW
CHUNK = 64
NCHUNK = BPW // CHUNK


def _mlp_body(t_ref, w1_ref, b1_ref, w2_ref, b2_ref, f_ref):
    e = t_ref[...].astype(jnp.bfloat16)
    h = jnp.dot(e, w1_ref[...].astype(jnp.bfloat16),
                preferred_element_type=jnp.float32)
    h = h + b1_ref[...]
    h = h * jax.nn.sigmoid(h)
    o = jnp.dot(h.astype(jnp.bfloat16), w2_ref[...].astype(jnp.bfloat16),
                preferred_element_type=jnp.float32)
    f_ref[...] = o + b2_ref[...]


def _expand_body(feat_hbm, idx_hbm, out_hbm, idx_v, buf0, buf1,
                 gsem0, gsem1, ssem0, ssem1):
    wid = lax.axis_index("s") * NC + lax.axis_index("c")
    base = wid * BPW
    pltpu.sync_copy(idx_hbm.at[pl.ds(base, BPW)], idx_v)
    bufs = (buf0, buf1)
    gsems = (gsem0, gsem1)
    ssems = (ssem0, ssem1)
    gathers = [None, None]
    scatters = [None, None]
    gathers[0] = pltpu.async_copy(
        feat_hbm.at[idx_v.at[pl.ds(0, CHUNK)]], buf0, gsem0)
    for j in range(NCHUNK):
        nxt = (j + 1) % 2
        if j + 1 < NCHUNK:
            if scatters[nxt] is not None:
                scatters[nxt].wait()
                scatters[nxt] = None
            gathers[nxt] = pltpu.async_copy(
                feat_hbm.at[idx_v.at[pl.ds((j + 1) * CHUNK, CHUNK)]],
                bufs[nxt], gsems[nxt])
        gathers[j % 2].wait()
        scatters[j % 2] = pltpu.async_copy(
            bufs[j % 2], out_hbm.at[pl.ds(base + j * CHUNK, CHUNK)],
            ssems[j % 2])
    for s in scatters:
        if s is not None:
            s.wait()


_expand = pl.kernel(
    _expand_body,
    out_type=jax.ShapeDtypeStruct((BATCH, FEAT), jnp.float32),
    mesh=plsc.VectorSubcoreMesh(core_axis_name="c", subcore_axis_name="s"),
    scratch_types=[
        pltpu.VMEM((BPW,), jnp.int32),
        pltpu.VMEM((CHUNK, FEAT), jnp.float32),
        pltpu.VMEM((CHUNK, FEAT), jnp.float32),
        pltpu.SemaphoreType.DMA,
        pltpu.SemaphoreType.DMA,
        pltpu.SemaphoreType.DMA,
        pltpu.SemaphoreType.DMA,
    ],
)


def kernel(action_indices, emb_table, W1, b1, W2, b2):
    table = jnp.pad(emb_table, ((0, PAD_ACTIONS - NUM_ACTIONS), (0, 0)))
    feat = pl.pallas_call(
        _mlp_body,
        grid=(PAD_ACTIONS // MLP_BM,),
        in_specs=[
            pl.BlockSpec((MLP_BM, EMBED), lambda i: (i, 0)),
            pl.BlockSpec((EMBED, FEAT), lambda i: (0, 0)),
            pl.BlockSpec((1, FEAT), lambda i: (0, 0)),
            pl.BlockSpec((FEAT, FEAT), lambda i: (0, 0)),
            pl.BlockSpec((1, FEAT), lambda i: (0, 0)),
        ],
        out_specs=pl.BlockSpec((MLP_BM, FEAT), lambda i: (i, 0)),
        out_shape=jax.ShapeDtypeStruct((PAD_ACTIONS, FEAT), jnp.float32),
    )(table, W1, b1.reshape(1, FEAT), W2, b2.reshape(1, FEAT))
    return _expand(feat, action_indices.astype(jnp.int32))


# speedup vs baseline: 1.3819x; 1.0370x over previous
import jax
import jax.numpy as jnp
from jax import lax
from jax.experimental import pallas as pl
from jax.experimental.pallas import tpu as pltpu
from jax.experimental.pallas import tpu_sc as plsc

NUM_ACTIONS = 1000
PAD_ACTIONS = 1024
EMBED = 128
FEAT = 512
BATCH = 16384

MLP_BM = 256

NC, NS = 2, 16
NW = NC * NS
BPW = BATCH // NW
CHUNK = 32
NCHUNK = BPW // CHUNK


def _mlp_body(t_ref, w1_ref, b1_ref, w2_ref, b2_ref, f_ref):
    e = t_ref[...].astype(jnp.bfloat16)
    h = jnp.dot(e, w1_ref[...].astype(jnp.bfloat16),
                preferred_element_type=jnp.float32)
    h = h + b1_ref[...]
    h = h * jax.nn.sigmoid(h)
    o = jnp.dot(h.astype(jnp.bfloat16), w2_ref[...].astype(jnp.bfloat16),
                preferred_element_type=jnp.float32)
    f_ref[...] = o + b2_ref[...]


def _expand_body(feat_hbm, idx_hbm, out_hbm, idx_v, buf0, buf1,
                 gsem0, gsem1, ssem0, ssem1):
    wid = lax.axis_index("s") * NC + lax.axis_index("c")
    base = wid * BPW
    pltpu.sync_copy(idx_hbm.at[pl.ds(base, BPW)], idx_v)
    bufs = (buf0, buf1)
    gsems = (gsem0, gsem1)
    ssems = (ssem0, ssem1)
    gathers = [None, None]
    scatters = [None, None]
    gathers[0] = pltpu.async_copy(
        feat_hbm.at[idx_v.at[pl.ds(0, CHUNK)]], buf0, gsem0)
    for j in range(NCHUNK):
        nxt = (j + 1) % 2
        if j + 1 < NCHUNK:
            if scatters[nxt] is not None:
                scatters[nxt].wait()
                scatters[nxt] = None
            gathers[nxt] = pltpu.async_copy(
                feat_hbm.at[idx_v.at[pl.ds((j + 1) * CHUNK, CHUNK)]],
                bufs[nxt], gsems[nxt])
        gathers[j % 2].wait()
        scatters[j % 2] = pltpu.async_copy(
            bufs[j % 2], out_hbm.at[pl.ds(base + j * CHUNK, CHUNK)],
            ssems[j % 2])
    for s in scatters:
        if s is not None:
            s.wait()


_expand = pl.kernel(
    _expand_body,
    out_type=jax.ShapeDtypeStruct((BATCH, FEAT), jnp.float32),
    mesh=plsc.VectorSubcoreMesh(core_axis_name="c", subcore_axis_name="s"),
    scratch_types=[
        pltpu.VMEM((BPW,), jnp.int32),
        pltpu.VMEM((CHUNK, FEAT), jnp.float32),
        pltpu.VMEM((CHUNK, FEAT), jnp.float32),
        pltpu.SemaphoreType.DMA,
        pltpu.SemaphoreType.DMA,
        pltpu.SemaphoreType.DMA,
        pltpu.SemaphoreType.DMA,
    ],
)


def kernel(action_indices, emb_table, W1, b1, W2, b2):
    feat = pl.pallas_call(
        _mlp_body,
        grid=(PAD_ACTIONS // MLP_BM,),
        in_specs=[
            pl.BlockSpec((MLP_BM, EMBED), lambda i: (i, 0)),
            pl.BlockSpec((EMBED, FEAT), lambda i: (0, 0)),
            pl.BlockSpec((1, FEAT), lambda i: (0, 0)),
            pl.BlockSpec((FEAT, FEAT), lambda i: (0, 0)),
            pl.BlockSpec((1, FEAT), lambda i: (0, 0)),
        ],
        out_specs=pl.BlockSpec((MLP_BM, FEAT), lambda i: (i, 0)),
        out_shape=jax.ShapeDtypeStruct((PAD_ACTIONS, FEAT), jnp.float32),
    )(emb_table, W1, b1.reshape(1, FEAT), W2, b2.reshape(1, FEAT))
    return _expand(feat, action_indices.astype(jnp.int32))
